# baseline (speedup 1.0000x reference)
import functools

import jax
import jax.numpy as jnp
from jax import lax
from jax.experimental import pallas as pl
from jax.experimental.pallas import tpu as pltpu
from jax.experimental.pallas import tpu_sc as plsc

EDGE_BOOST_VAL = 3.0
L = 16
NB = 5


def _make_sc_kernel(B, V, K, NF):
    NC, NS = 2, 16
    NW = NC * NS
    ROWS = B // NW
    W = ((V + NB - 1) // NB + 127) // 128 * 128
    OFFS = tuple(h * W for h in range(NB))
    SIZES = tuple([W] * (NB - 1) + [V - (NB - 1) * W])
    assert SIZES[-1] > 0 and SIZES[-1] <= W
    HORDER = tuple(list(range(1, NB)) + [0])
    mesh = plsc.VectorSubcoreMesh(core_axis_name="c", subcore_axis_name="s")

    @functools.partial(
        pl.kernel,
        mesh=mesh,
        compiler_params=pltpu.CompilerParams(needs_layout_passes=False),
        out_type=jax.ShapeDtypeStruct((B, V), jnp.float32),
        scratch_types=(
            [pltpu.VMEM((SIZES[h],), jnp.float32) for h in range(NB)] + [
                pltpu.VMEM((ROWS, K), jnp.int32),
                pltpu.VMEM((K,), jnp.float32),
                pltpu.VMEM((NF,), jnp.int32),
                pltpu.VMEM((NF,), jnp.float32),
                pltpu.SemaphoreType.DMA,
            ]
            + [pltpu.SemaphoreType.DMA for _ in range(NB)]
            + [pltpu.SemaphoreType.DMA for _ in range(NB)]
        ),
    )
    def k(scores_hbm, allowed_hbm, nf_hbm, out_hbm, *scratch):
        bufs = list(scratch[:NB])
        ids_v, vals_v, nf_v, bval_v, idsem = scratch[NB:NB + 5]
        isems = list(scratch[NB + 5:NB + 5 + NB])
        osems = list(scratch[NB + 5 + NB:NB + 5 + 2 * NB])
        wid = lax.axis_index("s") * NC + lax.axis_index("c")

        chunks = [(r, h) for r in range(ROWS) for h in HORDER]
        in_copies = [None] * NB
        out_copies = [None] * NB

        def fire_in(c):
            r, h = chunks[c]
            b = wid * ROWS + r
            in_copies[h] = pltpu.async_copy(
                scores_hbm.at[b].at[pl.ds(OFFS[h], SIZES[h])],
                bufs[h], isems[h])

        for c in range(NB - 1):
            fire_in(c)
        id_copies = [
            pltpu.async_copy(allowed_hbm.at[wid * ROWS + r], ids_v.at[r], idsem)
            for r in range(ROWS)
        ]

        pltpu.sync_copy(nf_hbm, nf_v)
        iota = lax.iota(jnp.int32, L)
        for h in range(NB):
            for j in range(NF // L):
                idx = nf_v[pl.ds(j * L, L)]
                tag = (iota + (j * L + 1)).astype(jnp.float32)
                m = (idx >= OFFS[h]) & (idx < OFFS[h] + SIZES[h])
                plsc.store_scatter(bufs[0], [jnp.where(m, idx - OFFS[h], 0)],
                                   tag, mask=m)
            for j in range(NF // L):
                idx = nf_v[pl.ds(j * L, L)]
                tag = (iota + (j * L + 1)).astype(jnp.float32)
                m = (idx >= OFFS[h]) & (idx < OFFS[h] + SIZES[h])
                back = plsc.load_gather(bufs[0],
                                        [jnp.where(m, idx - OFFS[h], 0)])
                bv = jnp.where(m & (back == tag),
                               jnp.float32(EDGE_BOOST_VAL), jnp.float32(0.0))
                if h == 0:
                    bval_v[pl.ds(j * L, L)] = bv
                else:
                    bval_v[pl.ds(j * L, L)] = bval_v[pl.ds(j * L, L)] + bv

        fire_in(NB - 1)
        for cp in id_copies:
            cp.wait()

        neg_inf = jnp.full((L,), float("-inf"), jnp.float32)
        for c, (r, h) in enumerate(chunks):
            b = wid * ROWS + r
            lo, hsz = OFFS[h], SIZES[h]
            buf = bufs[h]

            cn = c + NB - 1
            if NB <= cn < len(chunks):
                hn = chunks[cn][1]
                if out_copies[hn] is not None:
                    out_copies[hn].wait()
                fire_in(cn)

            in_copies[h].wait()

            def gather_body(j, _):
                ids = ids_v[r, pl.ds(j * L, L)]
                m = (ids >= lo) & (ids < lo + hsz)
                gidx = jnp.where(m, ids - lo, 0)
                vals_v[pl.ds(j * L, L)] = plsc.load_gather(buf, [gidx])
                return 0
            lax.fori_loop(0, K // L, gather_body, 0)

            def fill_body(i, _):
                buf[pl.ds(i * L, L)] = neg_inf
                return 0
            lax.fori_loop(0, hsz // L, fill_body, 0, unroll=8)

            def scatter_body(j, _):
                ids = ids_v[r, pl.ds(j * L, L)]
                val = vals_v[pl.ds(j * L, L)]
                m = (ids >= lo) & (ids < lo + hsz)
                plsc.store_scatter(buf, [ids - lo], val, mask=m)
                return 0
            lax.fori_loop(0, K // L, scatter_body, 0)

            def boost_body(j, _):
                idx = nf_v[pl.ds(j * L, L)]
                bv = bval_v[pl.ds(j * L, L)]
                m = (idx >= lo) & (idx < lo + hsz)
                plsc.addupdate_scatter(buf, [idx - lo], bv, mask=m)
                return 0
            lax.fori_loop(0, NF // L, boost_body, 0)

            out_copies[h] = pltpu.async_copy(
                buf, out_hbm.at[b].at[pl.ds(lo, hsz)], osems[h])

        for h in range(NB):
            if out_copies[h] is not None:
                out_copies[h].wait()

    return k


def kernel(input_ids, scores, allowed_ids, next_for_ids):
    del input_ids
    B, V = scores.shape
    K = allowed_ids.shape[1]
    NF = next_for_ids.shape[0]
    k = _make_sc_kernel(B, V, K, NF)
    return k(scores, allowed_ids, next_for_ids)


# speedup vs baseline: 1.1789x; 1.1789x over previous
import functools

import jax
import jax.numpy as jnp
from jax import lax
from jax.experimental import pallas as pl
from jax.experimental.pallas import tpu as pltpu
from jax.experimental.pallas import tpu_sc as plsc

EDGE_BOOST_VAL = 3.0
L = 16
NB = 2


def _make_sc_kernel(B, V, K, NF):
    NC, NS = 2, 16
    NW = NC * NS
    ROWS = B // NW
    W = ((V + NB - 1) // NB + 127) // 128 * 128
    OFFS = tuple(h * W for h in range(NB))
    SIZES = tuple([W] * (NB - 1) + [V - (NB - 1) * W])
    assert SIZES[-1] > 0 and SIZES[-1] <= W
    HORDER = tuple(list(range(1, NB)) + [0])
    mesh = plsc.VectorSubcoreMesh(core_axis_name="c", subcore_axis_name="s")

    @functools.partial(
        pl.kernel,
        mesh=mesh,
        compiler_params=pltpu.CompilerParams(needs_layout_passes=False),
        out_type=jax.ShapeDtypeStruct((B, V), jnp.float32),
        scratch_types=(
            [pltpu.VMEM((SIZES[h],), jnp.float32) for h in range(NB)] + [
                pltpu.VMEM((ROWS, K), jnp.int32),
                pltpu.VMEM((K,), jnp.float32),
                pltpu.VMEM((NF,), jnp.int32),
                pltpu.VMEM((NF,), jnp.float32),
                pltpu.SemaphoreType.DMA,
            ]
            + [pltpu.SemaphoreType.DMA for _ in range(NB)]
            + [pltpu.SemaphoreType.DMA for _ in range(NB)]
        ),
    )
    def k(scores_hbm, allowed_hbm, nf_hbm, out_hbm, *scratch):
        bufs = list(scratch[:NB])
        ids_v, vals_v, nf_v, bval_v, idsem = scratch[NB:NB + 5]
        isems = list(scratch[NB + 5:NB + 5 + NB])
        osems = list(scratch[NB + 5 + NB:NB + 5 + 2 * NB])
        wid = lax.axis_index("s") * NC + lax.axis_index("c")

        chunks = [(r, h) for r in range(ROWS) for h in HORDER]
        in_copies = [None] * NB
        out_copies = [None] * NB

        def fire_in(c):
            r, h = chunks[c]
            b = wid * ROWS + r
            in_copies[h] = pltpu.async_copy(
                scores_hbm.at[b].at[pl.ds(OFFS[h], SIZES[h])],
                bufs[h], isems[h])

        for c in range(NB - 1):
            fire_in(c)
        id_copies = [
            pltpu.async_copy(allowed_hbm.at[wid * ROWS + r], ids_v.at[r], idsem)
            for r in range(ROWS)
        ]

        pltpu.sync_copy(nf_hbm, nf_v)
        iota = lax.iota(jnp.int32, L)
        for h in range(NB):
            for j in range(NF // L):
                idx = nf_v[pl.ds(j * L, L)]
                tag = (iota + (j * L + 1)).astype(jnp.float32)
                m = (idx >= OFFS[h]) & (idx < OFFS[h] + SIZES[h])
                plsc.store_scatter(bufs[0], [jnp.where(m, idx - OFFS[h], 0)],
                                   tag, mask=m)
            for j in range(NF // L):
                idx = nf_v[pl.ds(j * L, L)]
                tag = (iota + (j * L + 1)).astype(jnp.float32)
                m = (idx >= OFFS[h]) & (idx < OFFS[h] + SIZES[h])
                back = plsc.load_gather(bufs[0],
                                        [jnp.where(m, idx - OFFS[h], 0)])
                bv = jnp.where(m & (back == tag),
                               jnp.float32(EDGE_BOOST_VAL), jnp.float32(0.0))
                if h == 0:
                    bval_v[pl.ds(j * L, L)] = bv
                else:
                    bval_v[pl.ds(j * L, L)] = bval_v[pl.ds(j * L, L)] + bv

        fire_in(NB - 1)
        for cp in id_copies:
            cp.wait()

        neg_inf = jnp.full((L,), float("-inf"), jnp.float32)
        for c, (r, h) in enumerate(chunks):
            b = wid * ROWS + r
            lo, hsz = OFFS[h], SIZES[h]
            buf = bufs[h]

            cn = c + NB - 1
            if NB <= cn < len(chunks):
                hn = chunks[cn][1]
                if out_copies[hn] is not None:
                    out_copies[hn].wait()
                fire_in(cn)

            in_copies[h].wait()

            def gather_body(j, _):
                ids = ids_v[r, pl.ds(j * L, L)]
                m = (ids >= lo) & (ids < lo + hsz)
                gidx = jnp.where(m, ids - lo, 0)
                vals_v[pl.ds(j * L, L)] = plsc.load_gather(buf, [gidx])
                return 0
            lax.fori_loop(0, K // L, gather_body, 0)

            def fill_body(i, _):
                buf[pl.ds(i * L, L)] = neg_inf
                return 0
            lax.fori_loop(0, hsz // L, fill_body, 0, unroll=16)

            def scatter_body(j, _):
                ids = ids_v[r, pl.ds(j * L, L)]
                val = vals_v[pl.ds(j * L, L)]
                m = (ids >= lo) & (ids < lo + hsz)
                plsc.store_scatter(buf, [ids - lo], val, mask=m)
                return 0
            lax.fori_loop(0, K // L, scatter_body, 0)

            def boost_body(j, _):
                idx = nf_v[pl.ds(j * L, L)]
                bv = bval_v[pl.ds(j * L, L)]
                m = (idx >= lo) & (idx < lo + hsz) & (bv > jnp.float32(0.0))
                plsc.addupdate_scatter(buf, [idx - lo], bv, mask=m)
                return 0
            lax.fori_loop(0, NF // L, boost_body, 0)

            out_copies[h] = pltpu.async_copy(
                buf, out_hbm.at[b].at[pl.ds(lo, hsz)], osems[h])

        for h in range(NB):
            if out_copies[h] is not None:
                out_copies[h].wait()

    return k


def kernel(input_ids, scores, allowed_ids, next_for_ids):
    del input_ids
    B, V = scores.shape
    K = allowed_ids.shape[1]
    NF = next_for_ids.shape[0]
    k = _make_sc_kernel(B, V, K, NF)
    return k(scores, allowed_ids, next_for_ids)
